# baseline (speedup 1.0000x reference)
import functools

import jax
import jax.numpy as jnp
from jax import lax
from jax.experimental import pallas as pl
from jax.experimental.pallas import tpu as pltpu

_VOCAB = 8192


def _loss_body(idx_ref, tgt_ref, row_ref, out_ref, loss_ref, acc_ref, *, n_tokens, vocab):
    i = pl.program_id(0)
    row = row_ref[...]
    out_ref[...] = row
    m = jnp.max(row)
    s = jnp.sum(jnp.exp(row - m))
    t = tgt_ref[i]
    lane = lax.broadcasted_iota(jnp.int32, (1, 1, vocab), 2)
    x_t = jnp.sum(jnp.where(lane == t, row, 0.0))
    nll = jnp.log(s) + m - x_t

    @pl.when(i == 0)
    def _init():
        acc_ref[0] = 0.0

    acc_ref[0] += nll

    @pl.when(i == n_tokens - 1)
    def _fin():
        loss_ref[...] = jnp.full((1, 1), acc_ref[0] / n_tokens, dtype=jnp.float32)


@functools.partial(jax.jit, static_argnames=("interpret",))
def _fused(idx_flat, targets_flat, table, interpret=False):
    n_tokens = idx_flat.shape[0]
    vocab = table.shape[1]
    table3 = table.reshape(table.shape[0], 1, vocab)
    grid_spec = pltpu.PrefetchScalarGridSpec(
        num_scalar_prefetch=2,
        grid=(n_tokens,),
        in_specs=[
            pl.BlockSpec((1, 1, vocab), lambda i, idx_ref, tgt_ref: (idx_ref[i], 0, 0)),
        ],
        out_specs=[
            pl.BlockSpec((1, 1, vocab), lambda i, idx_ref, tgt_ref: (i, 0, 0)),
            pl.BlockSpec((1, 1), lambda i, idx_ref, tgt_ref: (0, 0)),
        ],
        scratch_shapes=[pltpu.SMEM((1,), jnp.float32)],
    )
    logits, loss = pl.pallas_call(
        functools.partial(_loss_body, n_tokens=n_tokens, vocab=vocab),
        grid_spec=grid_spec,
        out_shape=[
            jax.ShapeDtypeStruct((n_tokens, 1, vocab), jnp.float32),
            jax.ShapeDtypeStruct((1, 1), jnp.float32),
        ],
        interpret=interpret,
    )(idx_flat, targets_flat, table3)
    return logits.reshape(n_tokens, vocab), loss[0, 0]


def kernel(idx, targets, table):
    b, t = idx.shape
    idx_flat = idx.reshape(b * t).astype(jnp.int32)
    targets_flat = targets.reshape(b * t).astype(jnp.int32)
    logits_flat, loss = _fused(idx_flat, targets_flat, table)
    return logits_flat.reshape(b, t, table.shape[1]), loss


# speedup vs baseline: 4.1834x; 4.1834x over previous
import functools

import jax
import jax.numpy as jnp
from jax import lax
from jax.experimental import pallas as pl
from jax.experimental.pallas import tpu as pltpu

_G = 8


def _loss_body(idx_ref, tgt_ref, *rest, n_tokens, vocab, g):
    row_refs = rest[:g]
    out_ref, loss_ref, acc_ref = rest[g], rest[g + 1], rest[g + 2]
    i = pl.program_id(0)

    rows = jnp.concatenate([r[...] for r in row_refs], axis=0)
    out_ref[...] = rows

    m = jnp.max(rows, axis=2)
    s = jnp.sum(jnp.exp(rows - m[:, :, None]), axis=2)
    lse = jnp.log(s) + m

    lane128 = lax.broadcasted_iota(jnp.int32, (1, 1, 128), 2)
    nll_sum = jnp.sum(lse)
    for j in range(g):
        t = tgt_ref[i * g + j]
        t_base = pl.multiple_of((t // 128) * 128, 128)
        win = row_refs[j][0, 0, pl.ds(t_base, 128)].reshape(1, 1, 128)
        x_t = jnp.sum(jnp.where(lane128 == (t - t_base), win, 0.0))
        nll_sum = nll_sum - x_t

    @pl.when(i == 0)
    def _init():
        acc_ref[0] = 0.0

    acc_ref[0] += nll_sum

    @pl.when(i == n_tokens // g - 1)
    def _fin():
        loss_ref[...] = jnp.full((1, 1), acc_ref[0] / n_tokens, dtype=jnp.float32)


@functools.partial(jax.jit, static_argnames=("interpret",))
def _fused(idx_flat, targets_flat, table, interpret=False):
    n_tokens = idx_flat.shape[0]
    vocab = table.shape[1]
    g = _G
    table3 = table.reshape(table.shape[0], 1, vocab)

    def mk_in_spec(j):
        return pl.BlockSpec((1, 1, vocab),
                            lambda i, idx_ref, tgt_ref, j=j: (idx_ref[i * g + j], 0, 0))

    grid_spec = pltpu.PrefetchScalarGridSpec(
        num_scalar_prefetch=2,
        grid=(n_tokens // g,),
        in_specs=[mk_in_spec(j) for j in range(g)],
        out_specs=[
            pl.BlockSpec((g, 1, vocab), lambda i, idx_ref, tgt_ref: (i, 0, 0)),
            pl.BlockSpec((1, 1), lambda i, idx_ref, tgt_ref: (0, 0)),
        ],
        scratch_shapes=[pltpu.SMEM((1,), jnp.float32)],
    )
    logits, loss = pl.pallas_call(
        functools.partial(_loss_body, n_tokens=n_tokens, vocab=vocab, g=g),
        grid_spec=grid_spec,
        out_shape=[
            jax.ShapeDtypeStruct((n_tokens, 1, vocab), jnp.float32),
            jax.ShapeDtypeStruct((1, 1), jnp.float32),
        ],
        interpret=interpret,
    )(idx_flat, targets_flat, *([table3] * g))
    return logits.reshape(n_tokens, vocab), loss[0, 0]


def kernel(idx, targets, table):
    b, t = idx.shape
    idx_flat = idx.reshape(b * t).astype(jnp.int32)
    targets_flat = targets.reshape(b * t).astype(jnp.int32)
    logits_flat, loss = _fused(idx_flat, targets_flat, table)
    return logits_flat.reshape(b, t, table.shape[1]), loss
